# baseline (speedup 1.0000x reference)
import functools

import jax
import jax.numpy as jnp
from jax import lax
from jax.experimental import pallas as pl
from jax.experimental.pallas import tpu as pltpu
from jax.experimental.pallas import tpu_sc as plsc

_VOCAB = 100000
_HID = 64
_LANES = 128
_BATCH = 4
_SEQ = 512
_NROWS = _BATCH * _SEQ

_NC = 2
_NS = 16
_NW = _NC * _NS
_RPW = _NROWS // _NW

_BN = 1024


def _gather_pe_sc(ids_flat, pe128, emb128):
    mesh = plsc.VectorSubcoreMesh(core_axis_name="c", subcore_axis_name="s")

    @functools.partial(
        pl.kernel,
        mesh=mesh,
        out_type=jax.ShapeDtypeStruct((_NROWS, _LANES), jnp.float32),
        scratch_types=[
            pltpu.VMEM((_RPW,), jnp.int32),
            pltpu.VMEM((_RPW, _LANES), jnp.float32),
            pltpu.SemaphoreType.DMA,
        ],
    )
    def sc_kernel(ids_hbm, pe_hbm, emb_hbm, out_hbm, idx_v, rows_v, sem):
        wid = lax.axis_index("s") * _NC + lax.axis_index("c")
        base = wid * _RPW
        pltpu.sync_copy(ids_hbm.at[pl.ds(base, _RPW)], idx_v)
        pltpu.sync_copy(pe_hbm.at[pl.ds(base, _RPW)], rows_v)
        pltpu.async_copy(emb_hbm.at[idx_v], rows_v, sem, add=True).wait()
        pltpu.sync_copy(rows_v, out_hbm.at[pl.ds(base, _RPW)])

    return sc_kernel(ids_flat, pe128, emb128)


def _project_body(x_ref, w_ref, o_ref):
    o_ref[...] = lax.dot_general(
        x_ref[:, :_HID],
        w_ref[...],
        dimension_numbers=(((1,), (1,)), ((), ())),
        preferred_element_type=jnp.float32,
    )


def _project_tc(xpe, W):
    return pl.pallas_call(
        _project_body,
        grid=(pl.cdiv(_VOCAB, _BN),),
        in_specs=[
            pl.BlockSpec((_NROWS, _LANES), lambda i: (0, 0)),
            pl.BlockSpec((_BN, _HID), lambda i: (i, 0)),
        ],
        out_specs=pl.BlockSpec((_NROWS, _BN), lambda i: (0, i)),
        out_shape=jax.ShapeDtypeStruct((_NROWS, _VOCAB), jnp.float32),
        compiler_params=pltpu.CompilerParams(
            dimension_semantics=("arbitrary",),
        ),
    )(xpe, W)


def kernel(ids, emb, W, pe):
    ids_flat = ids.reshape(_NROWS)
    emb128 = jnp.pad(emb, ((0, 0), (0, _LANES - _HID)))
    pe128 = jnp.pad(jnp.tile(pe, (_BATCH, 1)), ((0, 0), (0, _LANES - _HID)))
    xpe = _gather_pe_sc(ids_flat, pe128, emb128)
    out = _project_tc(xpe, W)
    return out.reshape(_BATCH, _SEQ, _VOCAB)


# speedup vs baseline: 2.4070x; 2.4070x over previous
import functools

import jax
import jax.numpy as jnp
from jax import lax
from jax.experimental import pallas as pl
from jax.experimental.pallas import tpu as pltpu
from jax.experimental.pallas import tpu_sc as plsc

_VOCAB = 100000
_HID = 64
_LANES = 128
_BATCH = 4
_SEQ = 512
_NROWS = _BATCH * _SEQ

_NC = 2
_NS = 16
_NW = _NC * _NS
_RPW = _NROWS // _NW

_BT = 2048
_BM = 2048


def _transpose_pad_body(et_ref, o_ref):
    o_ref[:, : _HID] = et_ref[...].T
    o_ref[:, _HID:] = jnp.zeros((_BT, _LANES - _HID), jnp.float32)


def _transpose_pad_tc(embT):
    return pl.pallas_call(
        _transpose_pad_body,
        grid=(pl.cdiv(_VOCAB, _BT),),
        in_specs=[pl.BlockSpec((_HID, _BT), lambda j: (0, j))],
        out_specs=pl.BlockSpec((_BT, _LANES), lambda j: (j, 0)),
        out_shape=jax.ShapeDtypeStruct((_VOCAB, _LANES), jnp.float32),
        compiler_params=pltpu.CompilerParams(
            dimension_semantics=("arbitrary",),
        ),
    )(embT)


def _gather_pe_sc(ids_flat, pe128, emb128):
    mesh = plsc.VectorSubcoreMesh(core_axis_name="c", subcore_axis_name="s")

    @functools.partial(
        pl.kernel,
        mesh=mesh,
        out_type=jax.ShapeDtypeStruct((_NROWS, _LANES), jnp.float32),
        scratch_types=[
            pltpu.VMEM((_RPW,), jnp.int32),
            pltpu.VMEM((_RPW, _LANES), jnp.float32),
            pltpu.SemaphoreType.DMA,
        ],
        compiler_params=pltpu.CompilerParams(use_tc_tiling_on_sc=True),
    )
    def sc_kernel(ids_hbm, pe_hbm, emb_hbm, out_hbm, idx_v, rows_v, sem):
        wid = lax.axis_index("s") * _NC + lax.axis_index("c")
        base = wid * _RPW
        pltpu.sync_copy(ids_hbm.at[pl.ds(base, _RPW)], idx_v)
        pltpu.sync_copy(pe_hbm.at[pl.ds(base, _RPW)], rows_v)
        pltpu.async_copy(emb_hbm.at[idx_v], rows_v, sem, add=True).wait()
        pltpu.sync_copy(rows_v, out_hbm.at[pl.ds(base, _RPW)])

    return sc_kernel(ids_flat, pe128, emb128)


def _project_body(x_ref, wt_ref, o_ref):
    b = pl.program_id(1)
    xb = x_ref[pl.ds(b * _SEQ, _SEQ), :_HID]
    o_ref[...] = lax.dot_general(
        wt_ref[...],
        xb,
        dimension_numbers=(((0,), (1,)), ((), ())),
        preferred_element_type=jnp.float32,
    )[None]


def _project_tc(xpe, WT):
    return pl.pallas_call(
        _project_body,
        grid=(pl.cdiv(_VOCAB, _BM), _BATCH),
        in_specs=[
            pl.BlockSpec((_NROWS, _LANES), lambda j, b: (0, 0)),
            pl.BlockSpec((_HID, _BM), lambda j, b: (0, j)),
        ],
        out_specs=pl.BlockSpec((1, _BM, _SEQ), lambda j, b: (b, j, 0)),
        out_shape=jax.ShapeDtypeStruct((_BATCH, _VOCAB, _SEQ), jnp.float32),
        compiler_params=pltpu.CompilerParams(
            dimension_semantics=("arbitrary", "arbitrary"),
        ),
    )(xpe, WT)


def kernel(ids, emb, W, pe):
    ids_flat = ids.reshape(_NROWS)
    emb128 = _transpose_pad_tc(emb.T)
    pe128 = jnp.pad(jnp.tile(pe, (_BATCH, 1)), ((0, 0), (0, _LANES - _HID)))
    xpe = _gather_pe_sc(ids_flat, pe128, emb128)
    out_t = _project_tc(xpe, W.T)
    return jnp.transpose(out_t, (0, 2, 1))


# speedup vs baseline: 2.8286x; 1.1752x over previous
import functools

import jax
import jax.numpy as jnp
from jax import lax
from jax.experimental import pallas as pl
from jax.experimental.pallas import tpu as pltpu
from jax.experimental.pallas import tpu_sc as plsc

_VOCAB = 100000
_HID = 64
_LANES = 128
_BATCH = 4
_SEQ = 512
_NROWS = _BATCH * _SEQ

_NC = 2
_NS = 16
_NW = _NC * _NS
_RPW = _NROWS // _NW

_BT = 2048
_BM = 4096


def _transpose_pad_body(et_ref, o_ref):
    o_ref[:, : _HID] = et_ref[...].T
    o_ref[:, _HID:] = jnp.zeros((_BT, _LANES - _HID), jnp.float32)


def _transpose_pad_tc(embT):
    return pl.pallas_call(
        _transpose_pad_body,
        grid=(pl.cdiv(_VOCAB, _BT),),
        in_specs=[pl.BlockSpec((_HID, _BT), lambda j: (0, j))],
        out_specs=pl.BlockSpec((_BT, _LANES), lambda j: (j, 0)),
        out_shape=jax.ShapeDtypeStruct((_VOCAB, _LANES), jnp.float32),
        compiler_params=pltpu.CompilerParams(
            dimension_semantics=("arbitrary",),
        ),
    )(embT)


def _gather_pe_sc(ids_flat, pe128, emb128):
    mesh = plsc.VectorSubcoreMesh(core_axis_name="c", subcore_axis_name="s")

    @functools.partial(
        pl.kernel,
        mesh=mesh,
        out_type=jax.ShapeDtypeStruct((_NROWS, _LANES), jnp.float32),
        scratch_types=[
            pltpu.VMEM((_RPW,), jnp.int32),
            pltpu.VMEM((_RPW, _LANES), jnp.float32),
            pltpu.SemaphoreType.DMA,
        ],
        compiler_params=pltpu.CompilerParams(use_tc_tiling_on_sc=True),
    )
    def sc_kernel(ids_hbm, pe_hbm, emb_hbm, out_hbm, idx_v, rows_v, sem):
        wid = lax.axis_index("s") * _NC + lax.axis_index("c")
        base = wid * _RPW
        pltpu.sync_copy(ids_hbm.at[pl.ds(base, _RPW)], idx_v)
        pltpu.sync_copy(pe_hbm.at[pl.ds(base, _RPW)], rows_v)
        pltpu.async_copy(emb_hbm.at[idx_v], rows_v, sem, add=True).wait()
        pltpu.sync_copy(rows_v, out_hbm.at[pl.ds(base, _RPW)])

    return sc_kernel(ids_flat, pe128, emb128)


def _project_body(x_ref, wt_ref, o_ref):
    b = pl.program_id(1)
    xb = x_ref[pl.ds(b * _SEQ, _SEQ), :_HID]
    o_ref[...] = lax.dot_general(
        wt_ref[...],
        xb,
        dimension_numbers=(((0,), (1,)), ((), ())),
        preferred_element_type=jnp.float32,
    )[None]


def _project_tc(xpe, WT):
    return pl.pallas_call(
        _project_body,
        grid=(pl.cdiv(_VOCAB, _BM), _BATCH),
        in_specs=[
            pl.BlockSpec((_NROWS, _LANES), lambda j, b: (0, 0)),
            pl.BlockSpec((_HID, _BM), lambda j, b: (0, j)),
        ],
        out_specs=pl.BlockSpec((1, _BM, _SEQ), lambda j, b: (b, j, 0)),
        out_shape=jax.ShapeDtypeStruct((_BATCH, _VOCAB, _SEQ), jnp.float32),
        compiler_params=pltpu.CompilerParams(
            dimension_semantics=("arbitrary", "arbitrary"),
        ),
    )(xpe, WT)


def kernel(ids, emb, W, pe):
    ids_flat = ids.reshape(_NROWS)
    emb128 = _transpose_pad_tc(emb.T)
    pe128 = jnp.pad(jnp.tile(pe, (_BATCH, 1)), ((0, 0), (0, _LANES - _HID)))
    xpe = _gather_pe_sc(ids_flat, pe128, emb128)
    out_t = _project_tc(xpe, W.T)
    return jnp.transpose(out_t, (0, 2, 1))
